# baseline (speedup 1.0000x reference)
import jax
import jax.numpy as jnp
from jax.experimental import pallas as pl
from jax.experimental.pallas import tpu as pltpu


def _round_up(a, b):
    return (a + b - 1) // b * b


def _convtr_kernel(xp_ref, wa_ref, wb_ref, b_ref, o_ref):
    q = o_ref.shape[0]
    half = wa_ref.shape[1]
    x_q = xp_ref[1:q + 1, :]
    a = jnp.concatenate([x_q, xp_ref[0:q, :]], axis=1)
    b = jnp.concatenate([xp_ref[2:q + 2, :], x_q], axis=1)
    bias = b_ref[...]
    o_ref[:, :half] = jnp.dot(
        a, wa_ref[...], preferred_element_type=jnp.float32) + bias
    o_ref[:, half:] = jnp.dot(
        b, wb_ref[...], preferred_element_type=jnp.float32) + bias


def kernel(v, g, bias, x):
    c_in, c_out, k = v.shape
    n, _, l_in = x.shape
    s, pad = 8, 4
    l_out = (l_in - 1) * s - 2 * pad + k
    q_len = -(-l_out // s)

    norm = jnp.sqrt(jnp.sum(v * v, axis=(1, 2), keepdims=True))
    w = (g * v / norm)

    def taps(lo, hi):
        return w[:, :, lo:hi].transpose(0, 2, 1).reshape(c_in, (hi - lo) * c_out)

    wa = jnp.concatenate([taps(4, 8), taps(12, 16)], axis=0).astype(jnp.bfloat16)
    wb = jnp.concatenate([taps(0, 4), taps(8, 12)], axis=0).astype(jnp.bfloat16)
    bias_row = jnp.tile(bias.astype(jnp.float32), (4,))[None, :]

    l_pad = _round_up(q_len + 2, 8)
    xp = jnp.pad(x.transpose(0, 2, 1),
                 ((0, 0), (1, l_pad - l_in - 1), (0, 0))).astype(jnp.bfloat16)

    n_chunks = 4 if n % 4 == 0 else 1
    nc = n // n_chunks

    out = jnp.zeros((n, c_out, l_out), jnp.float32)
    for i in range(n_chunks):
        xp_i = jax.lax.slice_in_dim(xp, i * nc, (i + 1) * nc, axis=0)
        acc = pl.pallas_call(
            _convtr_kernel,
            out_shape=jax.ShapeDtypeStruct((nc, q_len, s * c_out), jnp.float32),
            grid=(nc,),
            in_specs=[
                pl.BlockSpec((None, l_pad, c_in), lambda b: (b, 0, 0)),
                pl.BlockSpec((2 * c_in, 4 * c_out), lambda b: (0, 0)),
                pl.BlockSpec((2 * c_in, 4 * c_out), lambda b: (0, 0)),
                pl.BlockSpec((1, 4 * c_out), lambda b: (0, 0)),
            ],
            out_specs=pl.BlockSpec((None, q_len, s * c_out), lambda b: (b, 0, 0)),
            compiler_params=pltpu.CompilerParams(
                dimension_semantics=("parallel",)),
        )(xp_i, wa, wb, bias_row)
        out_i = jnp.transpose(
            acc.reshape(nc, q_len * s, c_out)[:, :l_out, :], (0, 2, 1))
        out = jax.lax.dynamic_update_slice(out, out_i, (i * nc, 0, 0))
    return out


# speedup vs baseline: 2.6140x; 2.6140x over previous
import jax
import jax.numpy as jnp
from jax.experimental import pallas as pl
from jax.experimental.pallas import tpu as pltpu


def _round_up(a, b):
    return (a + b - 1) // b * b


def _convtr_kernel(xp_ref, wa_ref, wb_ref, b_ref, o_ref, nlc_ref):
    q = nlc_ref.shape[1] // 8
    c_out = o_ref.shape[0]
    x_q = xp_ref[1:q + 1, :]
    a = jnp.concatenate([x_q, xp_ref[0:q, :]], axis=1)
    b = jnp.concatenate([xp_ref[2:q + 2, :], x_q], axis=1)
    bias = b_ref[...]
    lo = jnp.dot(a, wa_ref[...], preferred_element_type=jnp.float32) + bias
    hi = jnp.dot(b, wb_ref[...], preferred_element_type=jnp.float32) + bias
    n_half = c_out // 128
    for h in range(n_half):
        for p in range(4):
            c0 = p * c_out + h * 128
            nlc_ref[h, p:p + 8 * q:8, :] = lo[:, c0:c0 + 128]
            nlc_ref[h, p + 4:p + 4 + 8 * q:8, :] = hi[:, c0:c0 + 128]
        o_ref[h * 128:(h + 1) * 128, :] = jnp.transpose(nlc_ref[h], (1, 0))


def kernel(v, g, bias, x):
    c_in, c_out, k = v.shape
    n, _, l_in = x.shape
    s, pad = 8, 4
    l_out = (l_in - 1) * s - 2 * pad + k
    q_len = -(-l_out // s)

    norm = jnp.sqrt(jnp.sum(v * v, axis=(1, 2), keepdims=True))
    w = (g * v / norm)

    def taps(lo, hi):
        return w[:, :, lo:hi].transpose(0, 2, 1).reshape(c_in, (hi - lo) * c_out)

    wa = jnp.concatenate([taps(4, 8), taps(12, 16)], axis=0).astype(jnp.bfloat16)
    wb = jnp.concatenate([taps(0, 4), taps(8, 12)], axis=0).astype(jnp.bfloat16)
    bias_row = jnp.tile(bias.astype(jnp.float32), (4,))[None, :]

    l_pad = _round_up(q_len + 2, 8)
    xp = jnp.pad(x.transpose(0, 2, 1),
                 ((0, 0), (1, l_pad - l_in - 1), (0, 0))).astype(jnp.bfloat16)

    out = pl.pallas_call(
        _convtr_kernel,
        out_shape=jax.ShapeDtypeStruct((n, c_out, q_len * s), jnp.float32),
        grid=(n,),
        in_specs=[
            pl.BlockSpec((None, l_pad, c_in), lambda b: (b, 0, 0)),
            pl.BlockSpec((2 * c_in, 4 * c_out), lambda b: (0, 0)),
            pl.BlockSpec((2 * c_in, 4 * c_out), lambda b: (0, 0)),
            pl.BlockSpec((1, 4 * c_out), lambda b: (0, 0)),
        ],
        out_specs=pl.BlockSpec((None, c_out, q_len * s), lambda b: (b, 0, 0)),
        scratch_shapes=[pltpu.VMEM((c_out // 128, q_len * s, 128), jnp.float32)],
        compiler_params=pltpu.CompilerParams(
            dimension_semantics=("parallel",)),
    )(xp, wa, wb, bias_row)

    return out[:, :, :l_out]
